# baseline (speedup 1.0000x reference)
import functools

import jax
import jax.numpy as jnp
from jax import lax
from jax.experimental import pallas as pl
from jax.experimental.pallas import tpu as pltpu
from jax.experimental.pallas import tpu_sc as plsc

_NUM_CORES = 2
_NUM_SUBCORES = 16
_NUM_WORKERS = _NUM_CORES * _NUM_SUBCORES


def _sc_gather(wt, sidx, B, I, O, chunk):
    cpb = I // chunk
    n_chunks = B * cpb
    per_w = n_chunks // _NUM_WORKERS
    assert I % chunk == 0 and n_chunks % _NUM_WORKERS == 0 and chunk % 8 == 0

    mesh = plsc.VectorSubcoreMesh(core_axis_name="c", subcore_axis_name="s")

    @functools.partial(
        pl.kernel,
        out_type=jax.ShapeDtypeStruct((B, I, O), wt.dtype),
        mesh=mesh,
        scratch_types=[
            pltpu.VMEM((chunk,), jnp.int32),
            pltpu.VMEM((chunk, O), wt.dtype),
            pltpu.SemaphoreType.DMA,
        ],
    )
    def gather_kernel(wt_hbm, idx_hbm, g_hbm, idx_v, rows_v, sem):
        wid = lax.axis_index("s") * _NUM_CORES + lax.axis_index("c")
        base = wid * per_w

        def body(c, _):
            gc = base + c
            b = gc // cpb
            i0 = pl.multiple_of((gc % cpb) * chunk, 8)
            pltpu.sync_copy(idx_hbm.at[pl.ds(pl.multiple_of(gc * chunk, 8),
                                             chunk)], idx_v)
            pltpu.async_copy(wt_hbm.at[idx_v], rows_v, sem).wait()
            pltpu.sync_copy(rows_v, g_hbm.at[b, pl.ds(i0, chunk)])
            return _

        lax.fori_loop(0, per_w, body, 0)

    return gather_kernel(wt, sidx.reshape(B * I))


def _snn_half(w, xs, ii, I):
    nz = ii > 0
    wp = jnp.where(nz, pltpu.roll(w, 1, axis=0), 0.0)
    m = w * xs
    mp = jnp.where(nz, pltpu.roll(m, 1, axis=0), 0.0)
    ws = w + wp
    ms = m + mp
    d = jnp.clip(ws - 1.0, 1e-10, 1e10)
    cond = (ms > xs * d) & (ws > 1.0)
    key = jnp.where(cond, ii, I)
    imin = jnp.min(key, axis=0, keepdims=True)
    sel = ii == imin
    ms_sel = jnp.sum(jnp.where(sel, ms, 0.0), axis=0, keepdims=True)
    d_sel = jnp.sum(jnp.where(sel, d, 0.0), axis=0, keepdims=True)
    return jnp.where(imin == I, jnp.float32(1e10), ms_sel / d_sel)


def _dense_body(g_ref, xs_ref, o_ref, *, I, Oh):
    g = g_ref[0]
    w_lo = lax.bitcast_convert_type(g << 16, jnp.float32)
    w_hi = lax.bitcast_convert_type(g & jnp.int32(-65536), jnp.float32)
    xs = xs_ref[0]
    ii = lax.broadcasted_iota(jnp.int32, (I, Oh), 0)
    out_lo = _snn_half(w_lo, xs, ii, I)
    out_hi = _snn_half(w_hi, xs, ii, I)
    o_ref[0] = jnp.concatenate([out_lo, out_hi], axis=1)


def _tc_dense(g3, x_s3):
    B, I, Oh = g3.shape
    return pl.pallas_call(
        functools.partial(_dense_body, I=I, Oh=Oh),
        grid=(B,),
        in_specs=[
            pl.BlockSpec((1, I, Oh), lambda b: (b, 0, 0)),
            pl.BlockSpec((1, I, 1), lambda b: (b, 0, 0)),
        ],
        out_specs=pl.BlockSpec((1, 1, 2 * Oh), lambda b: (b, 0, 0)),
        out_shape=jax.ShapeDtypeStruct((B, 1, 2 * Oh), jnp.float32),
    )(g3, x_s3)


def kernel(input, W):
    B, I = input.shape
    O = W.shape[0]
    O_pad = ((O + 127) // 128) * 128
    Oh = O_pad // 2
    iota = jax.lax.broadcasted_iota(jnp.int32, (B, I), 1)
    x_s, sidx = jax.lax.sort((input, iota), dimension=1, num_keys=1,
                             is_stable=True)
    wtb = jnp.pad(W.T, ((0, 0), (0, O_pad - O))).astype(jnp.bfloat16)
    wt_pack = lax.bitcast_convert_type(
        jnp.stack([wtb[:, :Oh], wtb[:, Oh:]], axis=-1), jnp.int32)
    g = _sc_gather(wt_pack, sidx, B, I, Oh, chunk=112)
    out = _tc_dense(g, x_s.reshape(B, I, 1))
    return out.reshape(B, O_pad)[:, :O]


# speedup vs baseline: 1.0081x; 1.0081x over previous
import functools

import jax
import jax.numpy as jnp
from jax import lax
from jax.experimental import pallas as pl
from jax.experimental.pallas import tpu as pltpu
from jax.experimental.pallas import tpu_sc as plsc

_NUM_CORES = 2
_NUM_SUBCORES = 16
_NUM_WORKERS = _NUM_CORES * _NUM_SUBCORES


def _sc_gather(wt, sidx, B, I, O, chunk):
    cpb = I // chunk
    n_chunks = B * cpb
    per_w = n_chunks // _NUM_WORKERS
    assert I % chunk == 0 and n_chunks % _NUM_WORKERS == 0 and chunk % 8 == 0

    mesh = plsc.VectorSubcoreMesh(core_axis_name="c", subcore_axis_name="s")

    @functools.partial(
        pl.kernel,
        out_type=jax.ShapeDtypeStruct((B, I, O), wt.dtype),
        mesh=mesh,
        scratch_types=[
            pltpu.VMEM((chunk,), jnp.int32),
            pltpu.VMEM((chunk, O), wt.dtype),
            pltpu.SemaphoreType.DMA,
        ],
    )
    def gather_kernel(wt_hbm, idx_hbm, g_hbm, idx_v, rows_v, sem):
        wid = lax.axis_index("s") * _NUM_CORES + lax.axis_index("c")
        base = wid * per_w

        def body(c, _):
            gc = base + c
            b = gc // cpb
            i0 = pl.multiple_of((gc % cpb) * chunk, 8)
            pltpu.sync_copy(idx_hbm.at[pl.ds(pl.multiple_of(gc * chunk, 8),
                                             chunk)], idx_v)
            pltpu.async_copy(wt_hbm.at[idx_v], rows_v, sem).wait()
            pltpu.sync_copy(rows_v, g_hbm.at[b, pl.ds(i0, chunk)])
            return _

        lax.fori_loop(0, per_w, body, 0)

    return gather_kernel(wt, sidx.reshape(B * I))


def _snn_half(w, xs, xsp, ii, nz, I):
    wp = jnp.where(nz, pltpu.roll(w, 1, axis=0), 0.0)
    ws = w + wp
    ms = w * xs + wp * xsp
    d = jnp.maximum(ws - 1.0, 1e-10)
    cond = (ms > xs * d) & (ws > 1.0)
    key = jnp.where(cond, ii, I)
    imin = jnp.min(key, axis=0, keepdims=True)
    sel = ii == imin
    ms_sel = jnp.sum(jnp.where(sel, ms, 0.0), axis=0, keepdims=True)
    d_sel = jnp.sum(jnp.where(sel, d, 0.0), axis=0, keepdims=True)
    return jnp.where(imin == I, jnp.float32(1e10), ms_sel / d_sel)


def _dense_body(g_ref, xs_ref, xsp_ref, o_ref, *, I, Oh):
    g = g_ref[0]
    w_lo = lax.bitcast_convert_type(g << 16, jnp.float32)
    w_hi = lax.bitcast_convert_type(g & jnp.int32(-65536), jnp.float32)
    xs = xs_ref[0]
    xsp = xsp_ref[0]
    ii = lax.broadcasted_iota(jnp.int32, (I, Oh), 0)
    nz = ii > 0
    out_lo = _snn_half(w_lo, xs, xsp, ii, nz, I)
    out_hi = _snn_half(w_hi, xs, xsp, ii, nz, I)
    o_ref[0] = jnp.concatenate([out_lo, out_hi], axis=1)


def _tc_dense(g3, x_s3, x_sp3):
    B, I, Oh = g3.shape
    return pl.pallas_call(
        functools.partial(_dense_body, I=I, Oh=Oh),
        grid=(B,),
        in_specs=[
            pl.BlockSpec((1, I, Oh), lambda b: (b, 0, 0)),
            pl.BlockSpec((1, I, 1), lambda b: (b, 0, 0)),
            pl.BlockSpec((1, I, 1), lambda b: (b, 0, 0)),
        ],
        out_specs=pl.BlockSpec((1, 1, 2 * Oh), lambda b: (b, 0, 0)),
        out_shape=jax.ShapeDtypeStruct((B, 1, 2 * Oh), jnp.float32),
    )(g3, x_s3, x_sp3)


def kernel(input, W):
    B, I = input.shape
    O = W.shape[0]
    O_pad = ((O + 127) // 128) * 128
    Oh = O_pad // 2
    iota = jax.lax.broadcasted_iota(jnp.int32, (B, I), 1)
    x_s, sidx = jax.lax.sort((input, iota), dimension=1, num_keys=1,
                             is_stable=True)
    wtb = jnp.pad(W.T, ((0, 0), (0, O_pad - O))).astype(jnp.bfloat16)
    wt_pack = lax.bitcast_convert_type(
        jnp.stack([wtb[:, :Oh], wtb[:, Oh:]], axis=-1), jnp.int32)
    x_sp = jnp.concatenate([jnp.zeros((B, 1), jnp.float32), x_s[:, :-1]],
                           axis=1)
    K = 4
    Bc = B // K
    outs = []
    for k in range(K):
        sl = slice(k * Bc, (k + 1) * Bc)
        g = _sc_gather(wt_pack, sidx[sl], Bc, I, Oh, chunk=112)
        outs.append(_tc_dense(g, x_s[sl].reshape(Bc, I, 1),
                              x_sp[sl].reshape(Bc, I, 1)))
    out = jnp.concatenate(outs, axis=0)
    return out.reshape(B, O_pad)[:, :O]


# speedup vs baseline: 1.1908x; 1.1812x over previous
import functools

import jax
import jax.numpy as jnp
from jax import lax
from jax.experimental import pallas as pl
from jax.experimental.pallas import tpu as pltpu
from jax.experimental.pallas import tpu_sc as plsc

_NUM_CORES = 2
_NUM_SUBCORES = 16
_NUM_WORKERS = _NUM_CORES * _NUM_SUBCORES


def _sc_gather(wt, sidx, B, I, O, chunk):
    cpb = I // chunk
    n_chunks = B * cpb
    per_w = n_chunks // _NUM_WORKERS
    assert I % chunk == 0 and n_chunks % _NUM_WORKERS == 0 and chunk % 8 == 0

    mesh = plsc.VectorSubcoreMesh(core_axis_name="c", subcore_axis_name="s")

    @functools.partial(
        pl.kernel,
        out_type=jax.ShapeDtypeStruct((B, I, O), wt.dtype),
        mesh=mesh,
        scratch_types=[
            pltpu.VMEM((chunk,), jnp.int32),
            pltpu.VMEM((chunk, O), wt.dtype),
            pltpu.SemaphoreType.DMA,
        ],
    )
    def gather_kernel(wt_hbm, idx_hbm, g_hbm, idx_v, rows_v, sem):
        wid = lax.axis_index("s") * _NUM_CORES + lax.axis_index("c")
        base = wid * per_w

        def body(c, _):
            gc = base + c
            b = gc // cpb
            i0 = pl.multiple_of((gc % cpb) * chunk, 8)
            pltpu.sync_copy(idx_hbm.at[pl.ds(pl.multiple_of(gc * chunk, 8),
                                             chunk)], idx_v)
            pltpu.async_copy(wt_hbm.at[idx_v], rows_v, sem).wait()
            pltpu.sync_copy(rows_v, g_hbm.at[b, pl.ds(i0, chunk)])
            return _

        lax.fori_loop(0, per_w, body, 0)

    return gather_kernel(wt, sidx.reshape(B * I))


def _snn_half(w, xs, xsp, ii, nz, I):
    wp = jnp.where(nz, pltpu.roll(w, 1, axis=0), 0.0)
    ws = w + wp
    ms = w * xs + wp * xsp
    d = jnp.maximum(ws - 1.0, 1e-10)
    cond = (ms > xs * d) & (ws > 1.0)
    key = jnp.where(cond, ii, I)
    imin = jnp.min(key, axis=0, keepdims=True)
    sel = ii == imin
    ms_sel = jnp.sum(jnp.where(sel, ms, 0.0), axis=0, keepdims=True)
    d_sel = jnp.sum(jnp.where(sel, d, 0.0), axis=0, keepdims=True)
    return jnp.where(imin == I, jnp.float32(1e10), ms_sel / d_sel)


def _dense_body(g_ref, x2_ref, o_ref, *, I, Oh):
    g = g_ref[0]
    w_lo = lax.bitcast_convert_type(g << 16, jnp.float32)
    w_hi = lax.bitcast_convert_type(g & jnp.int32(-65536), jnp.float32)
    t = x2_ref[0]
    ones = jnp.ones((1, Oh), jnp.float32)
    dims = (((0,), (0,)), ((), ()))
    xs = lax.dot_general(t[0:1, :], ones, dims,
                         preferred_element_type=jnp.float32)
    xsp = lax.dot_general(t[1:2, :], ones, dims,
                          preferred_element_type=jnp.float32)
    ii = lax.broadcasted_iota(jnp.int32, (I, Oh), 0)
    nz = ii > 0
    out_lo = _snn_half(w_lo, xs, xsp, ii, nz, I)
    out_hi = _snn_half(w_hi, xs, xsp, ii, nz, I)
    o_ref[0] = jnp.concatenate([out_lo, out_hi], axis=1)


def _tc_dense(g3, x2):
    B, I, Oh = g3.shape
    return pl.pallas_call(
        functools.partial(_dense_body, I=I, Oh=Oh),
        grid=(B,),
        in_specs=[
            pl.BlockSpec((1, I, Oh), lambda b: (b, 0, 0)),
            pl.BlockSpec((1, 2, I), lambda b: (b, 0, 0)),
        ],
        out_specs=pl.BlockSpec((1, 1, 2 * Oh), lambda b: (b, 0, 0)),
        out_shape=jax.ShapeDtypeStruct((B, 1, 2 * Oh), jnp.float32),
    )(g3, x2)


def kernel(input, W):
    B, I = input.shape
    O = W.shape[0]
    O_pad = ((O + 127) // 128) * 128
    Oh = O_pad // 2
    iota = jax.lax.broadcasted_iota(jnp.int32, (B, I), 1)
    x_s, sidx = jax.lax.sort((input, iota), dimension=1, num_keys=1,
                             is_stable=True)
    wtb = jnp.pad(W.T, ((0, 0), (0, O_pad - O))).astype(jnp.bfloat16)
    wt_pack = lax.bitcast_convert_type(
        jnp.stack([wtb[:, :Oh], wtb[:, Oh:]], axis=-1), jnp.int32)
    x_sp = jnp.concatenate([jnp.zeros((B, 1), jnp.float32), x_s[:, :-1]],
                           axis=1)
    x2 = jnp.stack([x_s, x_sp], axis=1)
    K = 4
    Bc = B // K
    outs = []
    for k in range(K):
        sl = slice(k * Bc, (k + 1) * Bc)
        g = _sc_gather(wt_pack, sidx[sl], Bc, I, Oh, chunk=112)
        outs.append(_tc_dense(g, x2[sl]))
    out = jnp.concatenate(outs, axis=0)
    return out.reshape(B, O_pad)[:, :O]


# speedup vs baseline: 1.2730x; 1.0690x over previous
import functools

import jax
import jax.numpy as jnp
from jax import lax
from jax.experimental import pallas as pl
from jax.experimental.pallas import tpu as pltpu
from jax.experimental.pallas import tpu_sc as plsc

_NUM_CORES = 2
_NUM_SUBCORES = 16
_NUM_WORKERS = _NUM_CORES * _NUM_SUBCORES


def _sc_gather(wt, sidx, B, I, O, chunk):
    cpb = I // chunk
    n_chunks = B * cpb
    per_w = n_chunks // _NUM_WORKERS
    assert I % chunk == 0 and n_chunks % _NUM_WORKERS == 0 and chunk % 8 == 0

    mesh = plsc.VectorSubcoreMesh(core_axis_name="c", subcore_axis_name="s")

    @functools.partial(
        pl.kernel,
        out_type=jax.ShapeDtypeStruct((B, I, O), wt.dtype),
        mesh=mesh,
        scratch_types=[
            pltpu.VMEM((chunk,), jnp.int32),
            pltpu.VMEM((chunk, O), wt.dtype),
            pltpu.SemaphoreType.DMA,
        ],
    )
    def gather_kernel(wt_hbm, idx_hbm, g_hbm, idx_v, rows_v, sem):
        wid = lax.axis_index("s") * _NUM_CORES + lax.axis_index("c")
        base = wid * per_w

        def body(c, _):
            gc = base + c
            b = gc // cpb
            i0 = pl.multiple_of((gc % cpb) * chunk, 8)
            pltpu.sync_copy(idx_hbm.at[pl.ds(pl.multiple_of(gc * chunk, 8),
                                             chunk)], idx_v)
            pltpu.async_copy(wt_hbm.at[idx_v], rows_v, sem).wait()
            pltpu.sync_copy(rows_v, g_hbm.at[b, pl.ds(i0, chunk)])
            return _

        lax.fori_loop(0, per_w, body, 0)

    return gather_kernel(wt, sidx.reshape(B * I))


def _snn_half(w, xs, xsp, ii, nz, I):
    wp = jnp.where(nz, pltpu.roll(w, 1, axis=0), 0.0)
    ws = w + wp
    ms = w * xs + wp * xsp
    d = jnp.maximum(ws - 1.0, 1e-10)
    cond = (ms > xs * d) & (ws > 1.0)
    key = jnp.where(cond, ii, I)
    imin = jnp.min(key, axis=0, keepdims=True)
    sel = ii == imin
    ms_sel = jnp.sum(jnp.where(sel, ms, 0.0), axis=0, keepdims=True)
    d_sel = jnp.sum(jnp.where(sel, d, 0.0), axis=0, keepdims=True)
    return jnp.where(imin == I, jnp.float32(1e10), ms_sel / d_sel)


def _dense_body(g_ref, x2_ref, o_ref, *, I, Oh):
    g = g_ref[0]
    w_lo = lax.bitcast_convert_type(g << 16, jnp.float32)
    w_hi = lax.bitcast_convert_type(g & jnp.int32(-65536), jnp.float32)
    t = x2_ref[0]
    ones = jnp.ones((1, Oh), jnp.float32)
    dims = (((0,), (0,)), ((), ()))
    xs = lax.dot_general(t[0:1, :], ones, dims,
                         preferred_element_type=jnp.float32)
    xsp = lax.dot_general(t[1:2, :], ones, dims,
                          preferred_element_type=jnp.float32)
    ii = lax.broadcasted_iota(jnp.int32, (I, Oh), 0)
    nz = ii > 0
    out_lo = _snn_half(w_lo, xs, xsp, ii, nz, I)
    out_hi = _snn_half(w_hi, xs, xsp, ii, nz, I)
    o_ref[0] = jnp.concatenate([out_lo, out_hi], axis=1)


def _tc_dense(g3, x2):
    B, I, Oh = g3.shape
    return pl.pallas_call(
        functools.partial(_dense_body, I=I, Oh=Oh),
        grid=(B,),
        in_specs=[
            pl.BlockSpec((1, I, Oh), lambda b: (b, 0, 0)),
            pl.BlockSpec((1, 2, I), lambda b: (b, 0, 0)),
        ],
        out_specs=pl.BlockSpec((1, 1, 2 * Oh), lambda b: (b, 0, 0)),
        out_shape=jax.ShapeDtypeStruct((B, 1, 2 * Oh), jnp.float32),
    )(g3, x2)


def kernel(input, W):
    B, I = input.shape
    O = W.shape[0]
    O_pad = ((O + 127) // 128) * 128
    Oh = O_pad // 2
    wtb = jnp.pad(W.T, ((0, 0), (0, O_pad - O))).astype(jnp.bfloat16)
    wt_pack = lax.bitcast_convert_type(
        jnp.stack([wtb[:, :Oh], wtb[:, Oh:]], axis=-1), jnp.int32)
    K = 4
    Bc = B // K
    iota = jax.lax.broadcasted_iota(jnp.int32, (Bc, I), 1)
    outs = []
    for k in range(K):
        sl = slice(k * Bc, (k + 1) * Bc)
        x_s, sidx = jax.lax.sort((input[sl], iota), dimension=1, num_keys=1,
                                 is_stable=True)
        g = _sc_gather(wt_pack, sidx, Bc, I, Oh, chunk=112)
        x_sp = jnp.concatenate([jnp.zeros((Bc, 1), jnp.float32),
                                x_s[:, :-1]], axis=1)
        outs.append(_tc_dense(g, jnp.stack([x_s, x_sp], axis=1)))
    out = jnp.concatenate(outs, axis=0)
    return out.reshape(B, O_pad)[:, :O]


# speedup vs baseline: 1.2871x; 1.0111x over previous
import functools

import jax
import jax.numpy as jnp
from jax import lax
from jax.experimental import pallas as pl
from jax.experimental.pallas import tpu as pltpu
from jax.experimental.pallas import tpu_sc as plsc

_NUM_CORES = 2
_NUM_SUBCORES = 16
_NUM_WORKERS = _NUM_CORES * _NUM_SUBCORES


def _sc_gather(wt, sidx, B, I, O, chunk):
    cpb = I // chunk
    n_chunks = B * cpb
    per_w = n_chunks // _NUM_WORKERS
    assert I % chunk == 0 and n_chunks % _NUM_WORKERS == 0 and chunk % 8 == 0

    mesh = plsc.VectorSubcoreMesh(core_axis_name="c", subcore_axis_name="s")

    @functools.partial(
        pl.kernel,
        out_type=jax.ShapeDtypeStruct((B, I, O), wt.dtype),
        mesh=mesh,
        scratch_types=[
            pltpu.VMEM((chunk,), jnp.int32),
            pltpu.VMEM((chunk, O), wt.dtype),
            pltpu.SemaphoreType.DMA,
        ],
    )
    def gather_kernel(wt_hbm, idx_hbm, g_hbm, idx_v, rows_v, sem):
        wid = lax.axis_index("s") * _NUM_CORES + lax.axis_index("c")
        base = wid * per_w

        def body(c, _):
            gc = base + c
            b = gc // cpb
            i0 = pl.multiple_of((gc % cpb) * chunk, 8)
            pltpu.sync_copy(idx_hbm.at[pl.ds(pl.multiple_of(gc * chunk, 8),
                                             chunk)], idx_v)
            pltpu.async_copy(wt_hbm.at[idx_v], rows_v, sem).wait()
            pltpu.sync_copy(rows_v, g_hbm.at[b, pl.ds(i0, chunk)])
            return _

        lax.fori_loop(0, per_w, body, 0)

    return gather_kernel(wt, sidx.reshape(B * I))


def _snn_half(w, wp, xs, xsp, ii, I):
    ws = w + wp
    ms = w * xs + wp * xsp
    d = jnp.maximum(ws - 1.0, 1e-10)
    cond = (ms > xs * d) & (ws > 1.0)
    key = jnp.where(cond, ii, I)
    imin = jnp.min(key, axis=0, keepdims=True)
    sel = ii == imin
    ms_sel = jnp.sum(jnp.where(sel, ms, 0.0), axis=0, keepdims=True)
    d_sel = jnp.sum(jnp.where(sel, d, 0.0), axis=0, keepdims=True)
    return jnp.where(imin == I, jnp.float32(1e10), ms_sel / d_sel)


def _dense_body(g_ref, x2_ref, o_ref, *, I, Oh):
    g = g_ref[0]
    ii = lax.broadcasted_iota(jnp.int32, (I, Oh), 0)
    gp = jnp.where(ii > 0, pltpu.roll(g, 1, axis=0), 0)
    w_lo = lax.bitcast_convert_type(g << 16, jnp.float32)
    w_hi = lax.bitcast_convert_type(g & jnp.int32(-65536), jnp.float32)
    wp_lo = lax.bitcast_convert_type(gp << 16, jnp.float32)
    wp_hi = lax.bitcast_convert_type(gp & jnp.int32(-65536), jnp.float32)
    t = x2_ref[0]
    ones = jnp.ones((1, Oh), jnp.float32)
    dims = (((0,), (0,)), ((), ()))
    xs = lax.dot_general(t[0:1, :], ones, dims,
                         preferred_element_type=jnp.float32)
    xsp = lax.dot_general(t[1:2, :], ones, dims,
                          preferred_element_type=jnp.float32)
    out_lo = _snn_half(w_lo, wp_lo, xs, xsp, ii, I)
    out_hi = _snn_half(w_hi, wp_hi, xs, xsp, ii, I)
    o_ref[0] = jnp.concatenate([out_lo, out_hi], axis=1)


def _tc_dense(g3, x2):
    B, I, Oh = g3.shape
    return pl.pallas_call(
        functools.partial(_dense_body, I=I, Oh=Oh),
        grid=(B,),
        in_specs=[
            pl.BlockSpec((1, I, Oh), lambda b: (b, 0, 0)),
            pl.BlockSpec((1, 2, I), lambda b: (b, 0, 0)),
        ],
        out_specs=pl.BlockSpec((1, 1, 2 * Oh), lambda b: (b, 0, 0)),
        out_shape=jax.ShapeDtypeStruct((B, 1, 2 * Oh), jnp.float32),
    )(g3, x2)


def kernel(input, W):
    B, I = input.shape
    O = W.shape[0]
    O_pad = ((O + 127) // 128) * 128
    Oh = O_pad // 2
    wtb = jnp.pad(W.T, ((0, 0), (0, O_pad - O))).astype(jnp.bfloat16)
    wt_pack = lax.bitcast_convert_type(
        jnp.stack([wtb[:, :Oh], wtb[:, Oh:]], axis=-1), jnp.int32)
    K = 4
    Bc = B // K
    iota = jax.lax.broadcasted_iota(jnp.int32, (Bc, I), 1)
    outs = []
    for k in range(K):
        sl = slice(k * Bc, (k + 1) * Bc)
        x_s, sidx = jax.lax.sort((input[sl], iota), dimension=1, num_keys=1,
                                 is_stable=True)
        g = _sc_gather(wt_pack, sidx, Bc, I, Oh, chunk=112)
        x_sp = jnp.concatenate([jnp.zeros((Bc, 1), jnp.float32),
                                x_s[:, :-1]], axis=1)
        outs.append(_tc_dense(g, jnp.stack([x_s, x_sp], axis=1)))
    out = jnp.concatenate(outs, axis=0)
    return out.reshape(B, O_pad)[:, :O]


# speedup vs baseline: 1.3260x; 1.0302x over previous
import functools

import jax
import jax.numpy as jnp
from jax import lax
from jax.experimental import pallas as pl
from jax.experimental.pallas import tpu as pltpu
from jax.experimental.pallas import tpu_sc as plsc

_NUM_CORES = 2
_NUM_SUBCORES = 16
_NUM_WORKERS = _NUM_CORES * _NUM_SUBCORES


def _sc_gather(wt, sidx, B, I, O, chunk):
    cpb = I // chunk
    n_chunks = B * cpb
    per_w = n_chunks // _NUM_WORKERS
    assert I % chunk == 0 and n_chunks % _NUM_WORKERS == 0 and chunk % 8 == 0

    mesh = plsc.VectorSubcoreMesh(core_axis_name="c", subcore_axis_name="s")

    @functools.partial(
        pl.kernel,
        out_type=jax.ShapeDtypeStruct((B, I, O), wt.dtype),
        mesh=mesh,
        scratch_types=[
            pltpu.VMEM((chunk,), jnp.int32),
            pltpu.VMEM((chunk, O), wt.dtype),
            pltpu.SemaphoreType.DMA,
        ],
    )
    def gather_kernel(wt_hbm, idx_hbm, g_hbm, idx_v, rows_v, sem):
        wid = lax.axis_index("s") * _NUM_CORES + lax.axis_index("c")
        base = wid * per_w

        def body(c, _):
            gc = base + c
            b = gc // cpb
            i0 = pl.multiple_of((gc % cpb) * chunk, 8)
            pltpu.sync_copy(idx_hbm.at[pl.ds(pl.multiple_of(gc * chunk, 8),
                                             chunk)], idx_v)
            pltpu.async_copy(wt_hbm.at[idx_v], rows_v, sem).wait()
            pltpu.sync_copy(rows_v, g_hbm.at[b, pl.ds(i0, chunk)])
            return _

        lax.fori_loop(0, per_w, body, 0)

    return gather_kernel(wt, sidx.reshape(B * I))


def _snn_half(w, wp, xs, xsp, ii, I):
    ws = w + wp
    ms = w * xs + wp * xsp
    u = ws - 1.0
    cond = (ms > xs * u) & (ws > 1.0)
    key = jnp.where(cond, ii, I)
    imin = jnp.min(key, axis=0, keepdims=True)
    sel = ii == imin
    ms_sel = jnp.sum(jnp.where(sel, ms, 0.0), axis=0, keepdims=True)
    u_sel = jnp.sum(jnp.where(sel, u, 0.0), axis=0, keepdims=True)
    d_sel = jnp.maximum(u_sel, 1e-10)
    return jnp.where(imin == I, jnp.float32(1e10), ms_sel / d_sel)


def _dense_body(g_ref, x2_ref, o_ref, *, I, Oh):
    g = g_ref[0]
    ii = lax.broadcasted_iota(jnp.int32, (I, Oh), 0)
    gp = jnp.where(ii > 0, pltpu.roll(g, 1, axis=0), 0)
    w_lo = lax.bitcast_convert_type(g << 16, jnp.float32)
    w_hi = lax.bitcast_convert_type(g & jnp.int32(-65536), jnp.float32)
    wp_lo = lax.bitcast_convert_type(gp << 16, jnp.float32)
    wp_hi = lax.bitcast_convert_type(gp & jnp.int32(-65536), jnp.float32)
    t = x2_ref[0]
    ones = jnp.ones((1, Oh), jnp.float32)
    dims = (((0,), (0,)), ((), ()))
    xs = lax.dot_general(t[0:1, :], ones, dims,
                         preferred_element_type=jnp.float32)
    xsp = lax.dot_general(t[1:2, :], ones, dims,
                          preferred_element_type=jnp.float32)
    out_lo = _snn_half(w_lo, wp_lo, xs, xsp, ii, I)
    out_hi = _snn_half(w_hi, wp_hi, xs, xsp, ii, I)
    o_ref[0] = jnp.concatenate([out_lo, out_hi], axis=1)


def _tc_dense(g3, x2):
    B, I, Oh = g3.shape
    return pl.pallas_call(
        functools.partial(_dense_body, I=I, Oh=Oh),
        grid=(B,),
        in_specs=[
            pl.BlockSpec((1, I, Oh), lambda b: (b, 0, 0)),
            pl.BlockSpec((1, 2, I), lambda b: (b, 0, 0)),
        ],
        out_specs=pl.BlockSpec((1, 1, 2 * Oh), lambda b: (b, 0, 0)),
        out_shape=jax.ShapeDtypeStruct((B, 1, 2 * Oh), jnp.float32),
    )(g3, x2)


def kernel(input, W):
    B, I = input.shape
    O = W.shape[0]
    O_pad = ((O + 127) // 128) * 128
    Oh = O_pad // 2
    wtb = jnp.pad(W.T, ((0, 0), (0, O_pad - O))).astype(jnp.bfloat16)
    wt_pack = lax.bitcast_convert_type(
        jnp.stack([wtb[:, :Oh], wtb[:, Oh:]], axis=-1), jnp.int32)
    K = 4
    Bc = B // K
    iota = jax.lax.broadcasted_iota(jnp.int32, (Bc, I), 1)
    outs = []
    for k in range(K):
        sl = slice(k * Bc, (k + 1) * Bc)
        x_s, sidx = jax.lax.sort((input[sl], iota), dimension=1, num_keys=1,
                                 is_stable=True)
        g = _sc_gather(wt_pack, sidx, Bc, I, Oh, chunk=112)
        x_sp = jnp.concatenate([jnp.zeros((Bc, 1), jnp.float32),
                                x_s[:, :-1]], axis=1)
        outs.append(_tc_dense(g, jnp.stack([x_s, x_sp], axis=1)))
    out = jnp.concatenate(outs, axis=0)
    return out.reshape(B, O_pad)[:, :O]


# speedup vs baseline: 1.4796x; 1.1158x over previous
import functools

import jax
import jax.numpy as jnp
from jax import lax
from jax.experimental import pallas as pl
from jax.experimental.pallas import tpu as pltpu
from jax.experimental.pallas import tpu_sc as plsc

_NUM_CORES = 2
_NUM_SUBCORES = 16
_NUM_WORKERS = _NUM_CORES * _NUM_SUBCORES


def _sc_gather(wt, sidx, B, I, O, chunk):
    cpb = I // chunk
    n_chunks = B * cpb
    per_w = n_chunks // _NUM_WORKERS
    assert I % chunk == 0 and n_chunks % _NUM_WORKERS == 0 and chunk % 8 == 0

    mesh = plsc.VectorSubcoreMesh(core_axis_name="c", subcore_axis_name="s")

    @functools.partial(
        pl.kernel,
        out_type=jax.ShapeDtypeStruct((B, I, O), wt.dtype),
        mesh=mesh,
        scratch_types=[
            pltpu.VMEM((chunk,), jnp.int32),
            pltpu.VMEM((chunk, O), wt.dtype),
            pltpu.SemaphoreType.DMA,
        ],
    )
    def gather_kernel(wt_hbm, idx_hbm, g_hbm, idx_v, rows_v, sem):
        wid = lax.axis_index("s") * _NUM_CORES + lax.axis_index("c")
        base = wid * per_w

        def body(c, _):
            gc = base + c
            b = gc // cpb
            i0 = pl.multiple_of((gc % cpb) * chunk, 8)
            pltpu.sync_copy(idx_hbm.at[pl.ds(pl.multiple_of(gc * chunk, 8),
                                             chunk)], idx_v)
            pltpu.async_copy(wt_hbm.at[idx_v], rows_v, sem).wait()
            pltpu.sync_copy(rows_v, g_hbm.at[b, pl.ds(i0, chunk)])
            return _

        lax.fori_loop(0, per_w, body, 0)

    return gather_kernel(wt, sidx.reshape(B * I))


def _snn_half(w, wp, xs, xsp, ii, I):
    ws = w + wp
    ms = w * xs + wp * xsp
    u = ws - 1.0
    cond = (ms > xs * u) & (ws > 1.0)
    key = jnp.where(cond, ii, I)
    imin = jnp.min(key, axis=0, keepdims=True)
    sel = ii == imin
    ms_sel = jnp.sum(jnp.where(sel, ms, 0.0), axis=0, keepdims=True)
    u_sel = jnp.sum(jnp.where(sel, u, 0.0), axis=0, keepdims=True)
    d_sel = jnp.maximum(u_sel, 1e-10)
    return jnp.where(imin == I, jnp.float32(1e10), ms_sel / d_sel)


def _dense_body(g_ref, x2_ref, o_ref, *, I, Oh, R):
    ii = lax.broadcasted_iota(jnp.int32, (I, Oh), 0)
    ones = jnp.ones((1, Oh), jnp.float32)
    dims = (((0,), (0,)), ((), ()))
    for r in range(R):
        g = g_ref[r]
        gp = jnp.where(ii > 0, pltpu.roll(g, 1, axis=0), 0)
        w_lo = lax.bitcast_convert_type(g << 16, jnp.float32)
        w_hi = lax.bitcast_convert_type(g & jnp.int32(-65536), jnp.float32)
        wp_lo = lax.bitcast_convert_type(gp << 16, jnp.float32)
        wp_hi = lax.bitcast_convert_type(gp & jnp.int32(-65536), jnp.float32)
        t = x2_ref[r]
        xs = lax.dot_general(t[0:1, :], ones, dims,
                             preferred_element_type=jnp.float32)
        xsp = lax.dot_general(t[1:2, :], ones, dims,
                              preferred_element_type=jnp.float32)
        out_lo = _snn_half(w_lo, wp_lo, xs, xsp, ii, I)
        out_hi = _snn_half(w_hi, wp_hi, xs, xsp, ii, I)
        o_ref[r] = jnp.concatenate([out_lo, out_hi], axis=1)


def _tc_dense(g3, x2, rows_per_step=2):
    B, I, Oh = g3.shape
    R = rows_per_step
    return pl.pallas_call(
        functools.partial(_dense_body, I=I, Oh=Oh, R=R),
        grid=(B // R,),
        in_specs=[
            pl.BlockSpec((R, I, Oh), lambda b: (b, 0, 0)),
            pl.BlockSpec((R, 2, I), lambda b: (b, 0, 0)),
        ],
        out_specs=pl.BlockSpec((R, 1, 2 * Oh), lambda b: (b, 0, 0)),
        out_shape=jax.ShapeDtypeStruct((B, 1, 2 * Oh), jnp.float32),
    )(g3, x2)


def kernel(input, W):
    B, I = input.shape
    O = W.shape[0]
    O_pad = ((O + 127) // 128) * 128
    Oh = O_pad // 2
    wtb = jnp.pad(W.T, ((0, 0), (0, O_pad - O))).astype(jnp.bfloat16)
    wt_pack = lax.bitcast_convert_type(
        jnp.stack([wtb[:, :Oh], wtb[:, Oh:]], axis=-1), jnp.int32)
    K = 4
    Bc = B // K
    iota = jax.lax.broadcasted_iota(jnp.int32, (Bc, I), 1)
    outs = []
    for k in range(K):
        sl = slice(k * Bc, (k + 1) * Bc)
        x_s, sidx = jax.lax.sort((input[sl], iota), dimension=1, num_keys=1,
                                 is_stable=True)
        g = _sc_gather(wt_pack, sidx, Bc, I, Oh, chunk=112)
        x_sp = jnp.concatenate([jnp.zeros((Bc, 1), jnp.float32),
                                x_s[:, :-1]], axis=1)
        outs.append(_tc_dense(g, jnp.stack([x_s, x_sp], axis=1)))
    out = jnp.concatenate(outs, axis=0)
    return out.reshape(B, O_pad)[:, :O]


# speedup vs baseline: 1.5257x; 1.0312x over previous
import functools

import jax
import jax.numpy as jnp
from jax import lax
from jax.experimental import pallas as pl
from jax.experimental.pallas import tpu as pltpu
from jax.experimental.pallas import tpu_sc as plsc

_NUM_CORES = 2
_NUM_SUBCORES = 16
_NUM_WORKERS = _NUM_CORES * _NUM_SUBCORES


def _sc_gather(wt, sidx, B, I, O, chunk):
    cpb = I // chunk
    n_chunks = B * cpb
    per_w = n_chunks // _NUM_WORKERS
    assert I % chunk == 0 and n_chunks % _NUM_WORKERS == 0 and chunk % 8 == 0

    mesh = plsc.VectorSubcoreMesh(core_axis_name="c", subcore_axis_name="s")

    @functools.partial(
        pl.kernel,
        out_type=jax.ShapeDtypeStruct((B, I, O), wt.dtype),
        mesh=mesh,
        scratch_types=[
            pltpu.VMEM((chunk,), jnp.int32),
            pltpu.VMEM((chunk, O), wt.dtype),
            pltpu.SemaphoreType.DMA,
        ],
    )
    def gather_kernel(wt_hbm, idx_hbm, g_hbm, idx_v, rows_v, sem):
        wid = lax.axis_index("s") * _NUM_CORES + lax.axis_index("c")
        base = wid * per_w

        def body(c, _):
            gc = base + c
            b = gc // cpb
            i0 = pl.multiple_of((gc % cpb) * chunk, 8)
            pltpu.sync_copy(idx_hbm.at[pl.ds(pl.multiple_of(gc * chunk, 8),
                                             chunk)], idx_v)
            pltpu.async_copy(wt_hbm.at[idx_v], rows_v, sem).wait()
            pltpu.sync_copy(rows_v, g_hbm.at[b, pl.ds(i0, chunk)])
            return _

        lax.fori_loop(0, per_w, body, 0)

    return gather_kernel(wt, sidx.reshape(B * I))


def _snn_half(w, wp, xs, xsp, ii, I):
    ws = w + wp
    ms = w * xs + wp * xsp
    u = ws - 1.0
    cond = (ms > xs * u) & (ws > 1.0)
    key = jnp.where(cond, ii, I)
    imin = jnp.min(key, axis=0, keepdims=True)
    sel = ii == imin
    ms_sel = jnp.sum(jnp.where(sel, ms, 0.0), axis=0, keepdims=True)
    u_sel = jnp.sum(jnp.where(sel, u, 0.0), axis=0, keepdims=True)
    d_sel = jnp.maximum(u_sel, 1e-10)
    return jnp.where(imin == I, jnp.float32(1e10), ms_sel / d_sel)


def _dense_body(g_ref, x2_ref, o_ref, *, I, Oh, R):
    ii = lax.broadcasted_iota(jnp.int32, (I, Oh), 0)
    ones = jnp.ones((1, Oh), jnp.float32)
    dims = (((0,), (0,)), ((), ()))
    for r in range(R):
        g = g_ref[r]
        gp = jnp.where(ii > 0, pltpu.roll(g, 1, axis=0), 0)
        w_lo = lax.bitcast_convert_type(g << 16, jnp.float32)
        w_hi = lax.bitcast_convert_type(g & jnp.int32(-65536), jnp.float32)
        wp_lo = lax.bitcast_convert_type(gp << 16, jnp.float32)
        wp_hi = lax.bitcast_convert_type(gp & jnp.int32(-65536), jnp.float32)
        t = x2_ref[r]
        xs = lax.dot_general(t[0:1, :], ones, dims,
                             preferred_element_type=jnp.float32)
        xsp = lax.dot_general(t[1:2, :], ones, dims,
                              preferred_element_type=jnp.float32)
        out_lo = _snn_half(w_lo, wp_lo, xs, xsp, ii, I)
        out_hi = _snn_half(w_hi, wp_hi, xs, xsp, ii, I)
        o_ref[r] = jnp.concatenate([out_lo, out_hi], axis=1)


def _tc_dense(g3, x2, rows_per_step=4):
    B, I, Oh = g3.shape
    R = rows_per_step
    return pl.pallas_call(
        functools.partial(_dense_body, I=I, Oh=Oh, R=R),
        grid=(B // R,),
        in_specs=[
            pl.BlockSpec((R, I, Oh), lambda b: (b, 0, 0)),
            pl.BlockSpec((R, 2, I), lambda b: (b, 0, 0)),
        ],
        out_specs=pl.BlockSpec((R, 1, 2 * Oh), lambda b: (b, 0, 0)),
        out_shape=jax.ShapeDtypeStruct((B, 1, 2 * Oh), jnp.float32),
    )(g3, x2)


def kernel(input, W):
    B, I = input.shape
    O = W.shape[0]
    O_pad = ((O + 127) // 128) * 128
    Oh = O_pad // 2
    wtb = jnp.pad(W.T, ((0, 0), (0, O_pad - O))).astype(jnp.bfloat16)
    wt_pack = lax.bitcast_convert_type(
        jnp.stack([wtb[:, :Oh], wtb[:, Oh:]], axis=-1), jnp.int32)
    K = 4
    Bc = B // K
    iota = jax.lax.broadcasted_iota(jnp.int32, (Bc, I), 1)
    outs = []
    for k in range(K):
        sl = slice(k * Bc, (k + 1) * Bc)
        x_s, sidx = jax.lax.sort((input[sl], iota), dimension=1, num_keys=1,
                                 is_stable=True)
        g = _sc_gather(wt_pack, sidx, Bc, I, Oh, chunk=112)
        x_sp = jnp.concatenate([jnp.zeros((Bc, 1), jnp.float32),
                                x_s[:, :-1]], axis=1)
        outs.append(_tc_dense(g, jnp.stack([x_s, x_sp], axis=1)))
    out = jnp.concatenate(outs, axis=0)
    return out.reshape(B, O_pad)[:, :O]
